# baseline (speedup 1.0000x reference)
import numpy as np
import jax
import jax.numpy as jnp
from jax.experimental import pallas as pl
from jax.experimental.pallas import tpu as pltpu


TMR = 1024
TMC = 2048
NCACHE = 7

_ROW, _COL, _AR, _AC, _ZOFF, _CLOSE, _DTRICK, _SLOT, _FIN = range(9)


def _build_schedule(TR: int, TC: int, ncache: int) -> np.ndarray:
    ready_after = {c: 2 * c + 1 for c in range(TC)}
    steps = []
    for r in range(TR):
        cols = list(range(TC))
        dt = (r - 1) // 2 if r % 2 == 1 else -1
        if dt >= 0:
            cols.remove(dt)
            cols.append(dt)
        for k, c in enumerate(cols):
            close = 1 if k == TC - 1 else 0
            steps.append([r, c, r, c, 0, close,
                          1 if (close and c == dt) else 0, -1, 0])
    uppers = [(r, c) for r in range(TR) for c in range(TC)
              if r <= ready_after[c] and not (r % 2 == 1 and c == (r - 1) // 2)]
    slot_of = {t: k for k, t in enumerate(uppers[:ncache])}
    for s in steps:
        key = (s[_ROW], s[_COL])
        if key in slot_of:
            s[_SLOT] = slot_of[key]
    for (r, c) in uppers:
        if (r, c) in slot_of:
            steps.append([r, c, steps[TR * TC - 1][_AR],
                          steps[TR * TC - 1][_AC], TMR, 0, 0,
                          slot_of[(r, c)] + 1000, 0])
    for (r, c) in uppers:
        if (r, c) not in slot_of:
            steps.append([r, c, r, c, TMR, 0, 0, -1, 0])
    last_touch = {}
    for idx, s in enumerate(steps):
        last_touch[s[_ROW]] = idx
    for idx in last_touch.values():
        steps[idx][_FIN] = 1
    return np.asarray(steps, dtype=np.int32).T.copy()


def _dot(a, b):
    return jax.lax.dot_general(
        a, b, (((1,), (0,)), ((), ())),
        precision=jax.lax.Precision.DEFAULT,
        preferred_element_type=jnp.float32)


def _s1_body(x_ref, w1_ref, o_ref):
    o_ref[...] = _dot(x_ref[...], w1_ref[...])


def _gcn_body(meta_ref, s1_ref, adj_ref, b1_ref, w2_ref, b2_ref,
              out_ref, tab_ref, s2b_ref, z_ref, cache_ref):
    i = pl.program_id(0)
    n, nhid = s1_ref.shape
    nclass = w2_ref.shape[1]
    no = nhid + nclass
    r = meta_ref[_ROW, i]
    c = meta_ref[_COL, i]
    zoff = meta_ref[_ZOFF, i]
    close = meta_ref[_CLOSE, i]
    dtrick = meta_ref[_DTRICK, i]
    slot = meta_ref[_SLOT, i]
    fin = meta_ref[_FIN, i]
    bf = jnp.bfloat16

    @pl.when(i == 0)
    def _():
        tab_ref[...] = jnp.zeros_like(tab_ref)
        s2b_ref[...] = jnp.zeros_like(s2b_ref)
        z_ref[...] = jnp.zeros_like(z_ref)
        tab_ref[0:n, 0:nhid] = s1_ref[...]

    @pl.when(slot < 1000)
    def _():
        p = _dot(adj_ref[...], tab_ref[pl.ds(c * TMC, TMC), 0:no])
        z_ref[pl.ds(zoff, TMR), :] += p[:, :nhid]
        tab_ref[pl.ds(r * TMR, TMR), no:] += p[:, nhid:]

        @pl.when(slot >= 0)
        def _():
            cache_ref[pl.ds(slot * TMR, TMR), :] = adj_ref[...].astype(bf)

    @pl.when(slot >= 1000)
    def _():
        tab_ref[pl.ds(r * TMR, TMR), no:] += jnp.dot(
            cache_ref[pl.ds((slot - 1000) * TMR, TMR), :],
            s2b_ref[pl.ds(c * TMC, TMC), :],
            preferred_element_type=jnp.float32)

    @pl.when(close == 1)
    def _():
        h = jnp.maximum(z_ref[0:TMR, :] + b1_ref[...], 0.0)
        rows = jax.lax.broadcasted_iota(jnp.int32, h.shape, 0)
        h = jnp.where(rows < n - r * TMR, h, 0.0)
        s2b = _dot(h, w2_ref[...])
        tab_ref[pl.ds(r * TMR, TMR), nhid:no] = s2b
        s2b_ref[pl.ds(r * TMR, TMR), :] = s2b.astype(bf)
        z_ref[0:TMR, :] = jnp.zeros((TMR, nhid), jnp.float32)

        @pl.when(dtrick == 1)
        def _():
            tab_ref[pl.ds(r * TMR, TMR), no:] += _dot(adj_ref[:, TMR:], s2b)

    @pl.when(fin == 1)
    def _():
        out_ref[...] = tab_ref[pl.ds(r * TMR, TMR), no:] + b2_ref[...]


def kernel(x, adj, W1, b1, W2, b2):
    n, nfeat = x.shape
    nhid = W1.shape[1]
    nclass = W2.shape[1]
    TR = -(-n // TMR)
    TC = -(-n // TMC)
    npad = TR * TMR
    meta = jnp.asarray(_build_schedule(TR, TC, NCACHE))
    G = meta.shape[1]

    grid_spec = pltpu.PrefetchScalarGridSpec(
        num_scalar_prefetch=1,
        grid=(G,),
        in_specs=[
            pl.BlockSpec((n, nhid), lambda i, m: (0, 0)),
            pl.BlockSpec((TMR, TMC), lambda i, m: (m[_AR, i], m[_AC, i])),
            pl.BlockSpec((1, nhid), lambda i, m: (0, 0)),
            pl.BlockSpec((nhid, nclass), lambda i, m: (0, 0)),
            pl.BlockSpec((1, nclass), lambda i, m: (0, 0)),
        ],
        out_specs=pl.BlockSpec((TMR, nclass), lambda i, m: (m[_ROW, i], 0)),
        scratch_shapes=[
            pltpu.VMEM((npad, nhid + 2 * nclass), jnp.float32),
            pltpu.VMEM((npad, nclass), jnp.bfloat16),
            pltpu.VMEM((2 * TMR, nhid), jnp.float32),
            pltpu.VMEM((NCACHE * TMR, TMC), jnp.bfloat16),
        ],
    )
    s1 = pl.pallas_call(
        _s1_body,
        out_shape=jax.ShapeDtypeStruct((n, nhid), jnp.float32),
    )(x, W1)
    return pl.pallas_call(
        _gcn_body,
        grid_spec=grid_spec,
        out_shape=jax.ShapeDtypeStruct((n, nclass), jnp.float32),
        compiler_params=pltpu.CompilerParams(vmem_limit_bytes=67108864),
    )(meta, s1, adj, b1.reshape(1, nhid), W2, b2.reshape(1, nclass))


# speedup vs baseline: 1.0428x; 1.0428x over previous
import numpy as np
import jax
import jax.numpy as jnp
from jax.experimental import pallas as pl
from jax.experimental.pallas import tpu as pltpu


TM = 2048
NCACHE = 1

_ROW, _COL, _AR, _AC, _KIND, _CLOSE, _SLOT, _FIN = range(8)


def _build_schedule(T: int, ncache: int) -> np.ndarray:
    steps = []
    for r in range(T):
        for c in [j for j in range(T) if j != r] + [r]:
            steps.append([r, c, r, c, 0, 1 if c == r else 0, -1, 0])
    uppers = [(r, c) for r in range(T) for c in range(r + 1, T)]
    slot_of = {t: k for k, t in enumerate(uppers[:ncache])}
    for s in steps:
        key = (s[_ROW], s[_COL])
        if key in slot_of:
            s[_SLOT] = slot_of[key]
    for (r, c) in uppers:
        if (r, c) in slot_of:
            steps.append([r, c, T - 1, T - 1, 2, 0, slot_of[(r, c)], 0])
    for (r, c) in uppers:
        if (r, c) not in slot_of:
            steps.append([r, c, r, c, 1, 0, -1, 0])
    last_touch = {}
    for idx, s in enumerate(steps):
        last_touch[s[_ROW]] = idx
    for idx in last_touch.values():
        steps[idx][_FIN] = 1
    return np.asarray(steps, dtype=np.int32).T.copy()


def _dot(a, b):
    return jax.lax.dot_general(
        a, b, (((1,), (0,)), ((), ())),
        precision=jax.lax.Precision.DEFAULT,
        preferred_element_type=jnp.float32)


def _s1_body(x_ref, w1_ref, o_ref):
    o_ref[...] = _dot(x_ref[...], w1_ref[...])


def _gcn_body(meta_ref, s1_ref, adj_ref, b1_ref, w2_ref, b2_ref,
              out_ref, tab_ref, s2b_ref, z_ref, cache_ref):
    i = pl.program_id(0)
    n, nhid = s1_ref.shape
    nclass = w2_ref.shape[1]
    no = nhid + nclass
    r = meta_ref[_ROW, i]
    c = meta_ref[_COL, i]
    kind = meta_ref[_KIND, i]
    close = meta_ref[_CLOSE, i]
    slot = meta_ref[_SLOT, i]
    fin = meta_ref[_FIN, i]
    bf = jnp.bfloat16

    @pl.when(i == 0)
    def _():
        tab_ref[...] = jnp.zeros_like(tab_ref)
        s2b_ref[...] = jnp.zeros_like(s2b_ref)
        z_ref[...] = jnp.zeros_like(z_ref)
        tab_ref[0:n, 0:nhid] = s1_ref[...]

    @pl.when(kind == 0)
    def _():
        p = _dot(adj_ref[...], tab_ref[pl.ds(c * TM, TM), 0:no])
        z_ref[...] += p[:, :nhid]
        tab_ref[pl.ds(r * TM, TM), no:] += p[:, nhid:]

        @pl.when(slot >= 0)
        def _():
            cache_ref[pl.ds(slot * TM, TM), :] = adj_ref[...].astype(bf)

        @pl.when(close == 1)
        def _():
            h = jnp.maximum(z_ref[...] + b1_ref[...], 0.0)
            rows = jax.lax.broadcasted_iota(jnp.int32, h.shape, 0)
            h = jnp.where(rows < n - r * TM, h, 0.0)
            s2b = _dot(h, w2_ref[...])
            tab_ref[pl.ds(r * TM, TM), nhid:no] = s2b
            s2b_ref[pl.ds(r * TM, TM), :] = s2b.astype(bf)
            z_ref[...] = jnp.zeros_like(z_ref)
            tab_ref[pl.ds(r * TM, TM), no:] += _dot(adj_ref[...], s2b)

    @pl.when(kind == 1)
    def _():
        tab_ref[pl.ds(r * TM, TM), no:] += _dot(
            adj_ref[...], tab_ref[pl.ds(c * TM, TM), nhid:no])

    @pl.when(kind == 2)
    def _():
        tab_ref[pl.ds(r * TM, TM), no:] += jnp.dot(
            cache_ref[pl.ds(slot * TM, TM), :],
            s2b_ref[pl.ds(c * TM, TM), :],
            preferred_element_type=jnp.float32)

    @pl.when(fin == 1)
    def _():
        out_ref[...] = tab_ref[pl.ds(r * TM, TM), no:] + b2_ref[...]


def kernel(x, adj, W1, b1, W2, b2):
    n, nfeat = x.shape
    nhid = W1.shape[1]
    nclass = W2.shape[1]
    T = -(-n // TM)
    npad = T * TM
    meta = jnp.asarray(_build_schedule(T, NCACHE))
    G = meta.shape[1]

    grid_spec = pltpu.PrefetchScalarGridSpec(
        num_scalar_prefetch=1,
        grid=(G,),
        in_specs=[
            pl.BlockSpec((n, nhid), lambda i, m: (0, 0)),
            pl.BlockSpec((TM, TM), lambda i, m: (m[_AR, i], m[_AC, i])),
            pl.BlockSpec((1, nhid), lambda i, m: (0, 0)),
            pl.BlockSpec((nhid, nclass), lambda i, m: (0, 0)),
            pl.BlockSpec((1, nclass), lambda i, m: (0, 0)),
        ],
        out_specs=pl.BlockSpec((TM, nclass), lambda i, m: (m[_ROW, i], 0)),
        scratch_shapes=[
            pltpu.VMEM((npad, nhid + 2 * nclass), jnp.float32),
            pltpu.VMEM((npad, nclass), jnp.bfloat16),
            pltpu.VMEM((TM, nhid), jnp.float32),
            pltpu.VMEM((NCACHE * TM, TM), jnp.bfloat16),
        ],
    )
    s1 = pl.pallas_call(
        _s1_body,
        out_shape=jax.ShapeDtypeStruct((n, nhid), jnp.float32),
    )(x, W1)
    return pl.pallas_call(
        _gcn_body,
        grid_spec=grid_spec,
        out_shape=jax.ShapeDtypeStruct((n, nclass), jnp.float32),
        compiler_params=pltpu.CompilerParams(vmem_limit_bytes=67108864),
    )(meta, s1, adj, b1.reshape(1, nhid), W2, b2.reshape(1, nclass))


# speedup vs baseline: 1.0756x; 1.0315x over previous
import numpy as np
import jax
import jax.numpy as jnp
from jax.experimental import pallas as pl
from jax.experimental.pallas import tpu as pltpu


TM = 2048
NCACHE = 1

_ROW, _COL, _AR, _AC, _KIND, _CLOSE, _SLOT, _FIN = range(8)


def _build_schedule(T: int, ncache: int) -> np.ndarray:
    steps = []
    for r in range(T):
        for c in [j for j in range(T) if j != r] + [r]:
            steps.append([r, c, r, c, 0, 1 if c == r else 0, -1, 0])
    uppers = [(r, c) for r in range(T) for c in range(r + 1, T)]
    slot_of = {t: k for k, t in enumerate(uppers[:ncache])}
    for s in steps:
        key = (s[_ROW], s[_COL])
        if key in slot_of:
            s[_SLOT] = slot_of[key]
    for (r, c) in uppers:
        if (r, c) in slot_of:
            steps.append([r, c, T - 1, T - 1, 2, 0, slot_of[(r, c)], 0])
    for (r, c) in uppers:
        if (r, c) not in slot_of:
            steps.append([r, c, r, c, 1, 0, -1, 0])
    last_touch = {}
    for idx, s in enumerate(steps):
        last_touch[s[_ROW]] = idx
    for idx in last_touch.values():
        steps[idx][_FIN] = 1
    return np.asarray(steps, dtype=np.int32).T.copy()


def _dot(a, b):
    return jax.lax.dot_general(
        a, b, (((1,), (0,)), ((), ())),
        precision=jax.lax.Precision.DEFAULT,
        preferred_element_type=jnp.float32)


def _gcn_body(meta_ref, x_ref, w1_ref, adj_ref, b1_ref, w2_ref, b2_ref,
              out_ref, tab_ref, s2b_ref, z_ref, cache_ref):
    i = pl.program_id(0)
    n = x_ref.shape[0]
    nhid = w1_ref.shape[1]
    nclass = w2_ref.shape[1]
    no = nhid + nclass
    r = meta_ref[_ROW, i]
    c = meta_ref[_COL, i]
    kind = meta_ref[_KIND, i]
    close = meta_ref[_CLOSE, i]
    slot = meta_ref[_SLOT, i]
    fin = meta_ref[_FIN, i]
    bf = jnp.bfloat16

    @pl.when(i == 0)
    def _():
        tab_ref[...] = jnp.zeros_like(tab_ref)
        s2b_ref[...] = jnp.zeros_like(s2b_ref)
        z_ref[...] = jnp.zeros_like(z_ref)
        tab_ref[0:n, 0:nhid] = _dot(x_ref[...], w1_ref[...])

    @pl.when(kind == 0)
    def _():
        p = _dot(adj_ref[...], tab_ref[pl.ds(c * TM, TM), 0:no])
        z_ref[...] += p[:, :nhid]
        tab_ref[pl.ds(r * TM, TM), no:] += p[:, nhid:]

        @pl.when(slot >= 0)
        def _():
            cache_ref[pl.ds(slot * TM, TM), :] = adj_ref[...].astype(bf)

        @pl.when(close == 1)
        def _():
            h = jnp.maximum(z_ref[...] + b1_ref[...], 0.0)
            rows = jax.lax.broadcasted_iota(jnp.int32, h.shape, 0)
            h = jnp.where(rows < n - r * TM, h, 0.0)
            s2b = _dot(h, w2_ref[...])
            tab_ref[pl.ds(r * TM, TM), nhid:no] = s2b
            s2b_ref[pl.ds(r * TM, TM), :] = s2b.astype(bf)
            z_ref[...] = jnp.zeros_like(z_ref)
            tab_ref[pl.ds(r * TM, TM), no:] += _dot(adj_ref[...], s2b)

    @pl.when(kind == 1)
    def _():
        tab_ref[pl.ds(r * TM, TM), no:] += _dot(
            adj_ref[...], tab_ref[pl.ds(c * TM, TM), nhid:no])

    @pl.when(kind == 2)
    def _():
        tab_ref[pl.ds(r * TM, TM), no:] += jnp.dot(
            cache_ref[pl.ds(slot * TM, TM), :],
            s2b_ref[pl.ds(c * TM, TM), :],
            preferred_element_type=jnp.float32)

    @pl.when(fin == 1)
    def _():
        out_ref[...] = tab_ref[pl.ds(r * TM, TM), no:] + b2_ref[...]


def kernel(x, adj, W1, b1, W2, b2):
    n, nfeat = x.shape
    nhid = W1.shape[1]
    nclass = W2.shape[1]
    T = -(-n // TM)
    npad = T * TM
    meta = jnp.asarray(_build_schedule(T, NCACHE))
    G = meta.shape[1]

    grid_spec = pltpu.PrefetchScalarGridSpec(
        num_scalar_prefetch=1,
        grid=(G,),
        in_specs=[
            pl.BlockSpec((n, nfeat), lambda i, m: (0, 0)),
            pl.BlockSpec((nfeat, nhid), lambda i, m: (0, 0)),
            pl.BlockSpec((TM, TM), lambda i, m: (m[_AR, i], m[_AC, i])),
            pl.BlockSpec((1, nhid), lambda i, m: (0, 0)),
            pl.BlockSpec((nhid, nclass), lambda i, m: (0, 0)),
            pl.BlockSpec((1, nclass), lambda i, m: (0, 0)),
        ],
        out_specs=pl.BlockSpec((TM, nclass), lambda i, m: (m[_ROW, i], 0)),
        scratch_shapes=[
            pltpu.VMEM((npad, nhid + 2 * nclass), jnp.float32),
            pltpu.VMEM((npad, nclass), jnp.bfloat16),
            pltpu.VMEM((TM, nhid), jnp.float32),
            pltpu.VMEM((NCACHE * TM, TM), jnp.bfloat16),
        ],
    )
    return pl.pallas_call(
        _gcn_body,
        grid_spec=grid_spec,
        out_shape=jax.ShapeDtypeStruct((n, nclass), jnp.float32),
        compiler_params=pltpu.CompilerParams(vmem_limit_bytes=67108864),
    )(meta, x, W1, adj, b1.reshape(1, nhid), W2, b2.reshape(1, nclass))
